# baseline (speedup 1.0000x reference)
import functools

import jax
import jax.numpy as jnp
import numpy as np
from jax import lax
from jax.experimental import pallas as pl
from jax.experimental.pallas import tpu as pltpu
from jax.experimental.pallas import tpu_sc as plsc

VOCAB = 100000
D = 64
MAXLEN = 200
NG1 = 21
MAX_G = 3
BATCH = 4096
SEQLEN = 200
N = BATCH * SEQLEN


def _pe_table():
    pe = np.zeros((MAXLEN, D), dtype=np.float32)
    position = np.arange(MAXLEN, dtype=np.float32)[:, None]
    div_term = np.exp(np.arange(0, D, 2, dtype=np.float32) * (-np.log(10000.0) / D))
    pe[:, 0::2] = np.sin(position * div_term)
    pe[:, 1::2] = np.cos(position * div_term)
    return pe


_PE = _pe_table()


def _perm_matrix():
    p = np.zeros((D, D), dtype=np.float32)
    for c1 in range(2):
        for i in range(16):
            for j in range(2):
                p[32 * c1 + 16 * j + i, 32 * c1 + 2 * i + j] = 1.0
    return p


_PERM = _perm_matrix()


_R = 2000


def _fuse_body(tok_ref, gid_ref, gtab_ref, perm_ref, out_ref):
    gids = gid_ref[...]
    iota = lax.broadcasted_iota(jnp.int32, (_R, NG1), 1)
    counts = jnp.zeros((_R, NG1), jnp.float32)
    for g in range(MAX_G):
        gid_g = lax.slice(gids, (0, g), (_R, g + 1))
        counts = counts + (gid_g == iota).astype(jnp.float32)
    gavg = lax.dot_general(
        counts, gtab_ref[...], (((1,), (0,)), ((), ())),
        preferred_element_type=jnp.float32,
    )
    fused = tok_ref[...] + gavg * (1.0 / MAX_G)
    permuted = lax.dot_general(
        fused, perm_ref[...], (((1,), (0,)), ((), ())),
        preferred_element_type=jnp.float32,
    )
    out_ref[...] = permuted.astype(jnp.bfloat16)


def _build_fused(token_table, genre_table, token_to_genres):
    return pl.pallas_call(
        _fuse_body,
        grid=(VOCAB // _R,),
        in_specs=[
            pl.BlockSpec((_R, D), lambda i: (i, 0)),
            pl.BlockSpec((_R, MAX_G), lambda i: (i, 0)),
            pl.BlockSpec((NG1, D), lambda i: (0, 0)),
            pl.BlockSpec((D, D), lambda i: (0, 0)),
        ],
        out_specs=pl.BlockSpec((_R, D), lambda i: (i, 0)),
        out_shape=jax.ShapeDtypeStruct((VOCAB, D), jnp.bfloat16),
    )(token_table, token_to_genres, genre_table, jnp.asarray(_PERM))


_NW = 32
_BPW = BATCH // _NW
_CH = SEQLEN
_IW0 = 104
_IW1 = 96


def _gather_pe_body(fused_hbm, seq_hbm, pe_hbm, out_hbm,
                    idx_v, rows_v, res_v, pe_v, gsems, wsems):
    wid = lax.axis_index("s") * 2 + lax.axis_index("c")
    b0 = wid * _BPW
    pltpu.sync_copy(pe_hbm, pe_v)

    def load_idx(buf, c):
        pltpu.sync_copy(seq_hbm.at[b0 + c], idx_v.at[buf])

    def start_gathers(buf, c):
        pltpu.async_copy(
            fused_hbm.at[idx_v.at[buf, pl.ds(0, _IW0)]],
            rows_v.at[buf, pl.ds(0, _IW0)], gsems.at[buf])
        pltpu.async_copy(
            fused_hbm.at[idx_v.at[buf, pl.ds(_IW0, _IW1)]],
            rows_v.at[buf, pl.ds(_IW0, _IW1)], gsems.at[buf])

    def wait_gathers(buf):
        pltpu.make_async_copy(
            fused_hbm.at[pl.ds(0, _CH)], rows_v.at[buf], gsems.at[buf]).wait()

    def add_pe(buf):
        mask = jnp.int32(np.int32(-65536))

        @pl.loop(0, _CH)
        def _(j):
            for s in range(D // 32):
                v = rows_v[buf, j, pl.ds(s * 32, 32)]
                w = plsc.bitcast(v, jnp.int32)
                a = plsc.bitcast(jnp.left_shift(w, 16), jnp.float32)
                b = plsc.bitcast(jnp.bitwise_and(w, mask), jnp.float32)
                sla = pl.ds(s * 32, 16)
                slb = pl.ds(s * 32 + 16, 16)
                res_v[buf, j, sla] = a + pe_v[j, sla]
                res_v[buf, j, slb] = b + pe_v[j, slb]

    def start_write(buf, c):
        pltpu.async_copy(res_v.at[buf], out_hbm.at[b0 + c], wsems.at[buf])

    def wait_write(buf):
        pltpu.make_async_copy(
            res_v.at[buf], out_hbm.at[b0], wsems.at[buf]).wait()

    load_idx(0, 0)
    start_gathers(0, 0)
    load_idx(1, 1)
    start_gathers(1, 1)

    @pl.loop(0, _BPW - 2, step=2)
    def _(cc):
        for buf in range(2):
            wait_gathers(buf)
            add_pe(buf)
            start_write(buf, cc + buf)
        for buf in range(2):
            load_idx(buf, cc + 2 + buf)
            wait_write(buf)
            start_gathers(buf, cc + 2 + buf)

    for buf in range(2):
        wait_gathers(buf)
        add_pe(buf)
        start_write(buf, _BPW - 2 + buf)
    for buf in range(2):
        wait_write(buf)


@functools.cache
def _gather_pe():
    mesh = plsc.VectorSubcoreMesh(core_axis_name="c", subcore_axis_name="s")
    return pl.kernel(
        _gather_pe_body,
        out_type=jax.ShapeDtypeStruct((BATCH, SEQLEN, D), jnp.float32),
        mesh=mesh,
        scratch_types=[
            pltpu.VMEM((2, _CH), jnp.int32),
            pltpu.VMEM((2, _CH, D), jnp.bfloat16),
            pltpu.VMEM((2, _CH, D), jnp.float32),
            pltpu.VMEM((MAXLEN, D), jnp.float32),
            pltpu.SemaphoreType.DMA((2,)),
            pltpu.SemaphoreType.DMA((2,)),
        ],
        compiler_params=pltpu.CompilerParams(
            use_tc_tiling_on_sc=False, needs_layout_passes=False),
    )


def kernel(sequence, token_table, genre_table, token_to_genres):
    fused = _build_fused(token_table, genre_table, token_to_genres)
    pe = jnp.asarray(_PE)
    return _gather_pe()(fused, sequence, pe)


# speedup vs baseline: 1.2010x; 1.2010x over previous
import functools

import jax
import jax.numpy as jnp
import numpy as np
from jax import lax
from jax.experimental import pallas as pl
from jax.experimental.pallas import tpu as pltpu
from jax.experimental.pallas import tpu_sc as plsc

VOCAB = 100000
D = 64
MAXLEN = 200
NG1 = 21
MAX_G = 3
BATCH = 4096
SEQLEN = 200
N = BATCH * SEQLEN


def _pe_table():
    pe = np.zeros((MAXLEN, D), dtype=np.float32)
    position = np.arange(MAXLEN, dtype=np.float32)[:, None]
    div_term = np.exp(np.arange(0, D, 2, dtype=np.float32) * (-np.log(10000.0) / D))
    pe[:, 0::2] = np.sin(position * div_term)
    pe[:, 1::2] = np.cos(position * div_term)
    return pe


_PE = _pe_table()


_R = 2000


def _fuse_body(tok_ref, gid_ref, gtab_ref, out_ref):
    gids = gid_ref[...]
    iota = lax.broadcasted_iota(jnp.int32, (_R, NG1), 1)
    counts = jnp.zeros((_R, NG1), jnp.float32)
    for g in range(MAX_G):
        gid_g = lax.slice(gids, (0, g), (_R, g + 1))
        counts = counts + (gid_g == iota).astype(jnp.float32)
    gavg = lax.dot_general(
        counts, gtab_ref[...], (((1,), (0,)), ((), ())),
        preferred_element_type=jnp.float32,
    )
    out_ref[...] = tok_ref[...] + gavg * (1.0 / MAX_G)


def _build_fused(token_table, genre_table, token_to_genres):
    return pl.pallas_call(
        _fuse_body,
        grid=(VOCAB // _R,),
        in_specs=[
            pl.BlockSpec((_R, D), lambda i: (i, 0)),
            pl.BlockSpec((_R, MAX_G), lambda i: (i, 0)),
            pl.BlockSpec((NG1, D), lambda i: (0, 0)),
        ],
        out_specs=pl.BlockSpec((_R, D), lambda i: (i, 0)),
        out_shape=jax.ShapeDtypeStruct((VOCAB, D), jnp.float32),
    )(token_table, token_to_genres, genre_table)


_NW = 32
_BPW = BATCH // _NW
_CH = SEQLEN
_IW0 = 104
_IW1 = 96


def _gather_pe_body(fused_hbm, seq_hbm, pe_hbm, out_hbm,
                    idx_v, rows_v, pe_v, gsems, wsems):
    wid = lax.axis_index("s") * 2 + lax.axis_index("c")
    b0 = wid * _BPW
    pltpu.sync_copy(pe_hbm, pe_v)

    def load_idx(buf, c):
        pltpu.sync_copy(seq_hbm.at[b0 + c], idx_v.at[buf])

    def start_gathers(buf, c):
        pltpu.async_copy(
            fused_hbm.at[idx_v.at[buf, pl.ds(0, _IW0)]],
            rows_v.at[buf, pl.ds(0, _IW0)], gsems.at[buf])
        pltpu.async_copy(
            fused_hbm.at[idx_v.at[buf, pl.ds(_IW0, _IW1)]],
            rows_v.at[buf, pl.ds(_IW0, _IW1)], gsems.at[buf])

    def wait_gathers(buf):
        pltpu.make_async_copy(
            out_hbm.at[b0], rows_v.at[buf], gsems.at[buf]).wait()

    def add_pe(buf):
        @pl.loop(0, _CH)
        def _(j):
            for s in range(D // 16):
                sl = pl.ds(s * 16, 16)
                rows_v[buf, j, sl] = rows_v[buf, j, sl] + pe_v[j, sl]

    def start_write(buf, c):
        pltpu.async_copy(rows_v.at[buf], out_hbm.at[b0 + c], wsems.at[buf])

    def wait_write(buf):
        pltpu.make_async_copy(
            rows_v.at[buf], out_hbm.at[b0], wsems.at[buf]).wait()

    load_idx(0, 0)
    start_gathers(0, 0)
    load_idx(1, 1)
    start_gathers(1, 1)

    @pl.loop(0, _BPW - 2, step=2)
    def _(cc):
        for buf in range(2):
            wait_gathers(buf)
            add_pe(buf)
            start_write(buf, cc + buf)
        for buf in range(2):
            load_idx(buf, cc + 2 + buf)
            wait_write(buf)
            start_gathers(buf, cc + 2 + buf)

    for buf in range(2):
        wait_gathers(buf)
        add_pe(buf)
        start_write(buf, _BPW - 2 + buf)
    for buf in range(2):
        wait_write(buf)


@functools.cache
def _gather_pe():
    mesh = plsc.VectorSubcoreMesh(core_axis_name="c", subcore_axis_name="s")
    return pl.kernel(
        _gather_pe_body,
        out_type=jax.ShapeDtypeStruct((BATCH, SEQLEN, D), jnp.float32),
        mesh=mesh,
        scratch_types=[
            pltpu.VMEM((2, _CH), jnp.int32),
            pltpu.VMEM((2, _CH, D), jnp.float32),
            pltpu.VMEM((MAXLEN, D), jnp.float32),
            pltpu.SemaphoreType.DMA((2,)),
            pltpu.SemaphoreType.DMA((2,)),
        ],
        compiler_params=pltpu.CompilerParams(use_tc_tiling_on_sc=False),
    )


def kernel(sequence, token_table, genre_table, token_to_genres):
    fused = _build_fused(token_table, genre_table, token_to_genres)
    pe = jnp.asarray(_PE)
    return _gather_pe()(fused, sequence, pe)
